# baseline (speedup 1.0000x reference)
import functools

import jax
import jax.numpy as jnp
from jax import lax
from jax.experimental import pallas as pl
from jax.experimental.pallas import tpu as pltpu
from jax.experimental.pallas import tpu_sc as plsc

N = 10000
M = 10000
E = 320000
H = 128

_NC = 2
_NS = 16
_NW = _NC * _NS
_K = 80


def _sc_seg_sum_rows(table, gidx, sidx, num_out):
    e_total = gidx.shape[0]
    per_w = e_total // _NW
    n_chunks = per_w // _K
    rows_per_tile = ((num_out + _NS - 1) // _NS + 7) // 8 * 8
    num_pad = rows_per_tile * _NS
    mesh = plsc.VectorSubcoreMesh(core_axis_name="c", subcore_axis_name="s")
    zeros = jnp.zeros((_K, H), jnp.float32)

    chunks = []
    off = 0
    while off < rows_per_tile:
        ln = min(_K, rows_per_tile - off)
        chunks.append((off, ln))
        off += ln

    @functools.partial(
        pl.kernel,
        out_type=jax.ShapeDtypeStruct((_NC * num_pad, H), jnp.float32),
        mesh=mesh,
        scratch_types=[
            pltpu.VMEM((_K,), jnp.int32),
            pltpu.VMEM((_K,), jnp.int32),
            pltpu.VMEM((_K, H), jnp.float32),
            pltpu.VMEM_SHARED((num_pad, H), jnp.float32),
            pltpu.SemaphoreType.DMA,
        ],
    )
    def k(table_h, gidx_h, sidx_h, zero_h, out_h, gi_v, si_v, rows_v, acc_s, sem):
        cid = lax.axis_index("c")
        sid = lax.axis_index("s")
        wid = cid * _NS + sid
        row0 = sid * rows_per_tile
        pltpu.sync_copy(zero_h, rows_v)
        for off, ln in chunks:
            pltpu.sync_copy(rows_v.at[pl.ds(0, ln)], acc_s.at[pl.ds(row0 + off, ln)])
        plsc.subcore_barrier()
        base0 = wid * per_w

        def body(j, carry):
            base = base0 + j * _K
            pltpu.sync_copy(gidx_h.at[pl.ds(base, _K)], gi_v)
            pltpu.sync_copy(sidx_h.at[pl.ds(base, _K)], si_v)
            pltpu.async_copy(table_h.at[gi_v], rows_v, sem).wait()
            pltpu.sync_copy(rows_v, acc_s.at[si_v], add=True)
            return carry

        lax.fori_loop(0, n_chunks, body, 0)
        plsc.subcore_barrier()
        out0 = cid * num_pad + row0
        for off, ln in chunks:
            pltpu.sync_copy(acc_s.at[pl.ds(row0 + off, ln)], rows_v.at[pl.ds(0, ln)])
            pltpu.sync_copy(rows_v.at[pl.ds(0, ln)], out_h.at[pl.ds(out0 + off, ln)])

    out = k(table, gidx, sidx, zeros)
    return out.reshape(_NC, num_pad, H)


def _l2norm(x):
    n = jnp.linalg.norm(x, axis=1, keepdims=True)
    return x / jnp.maximum(n, 1e-12)


def _linear(x, W, b):
    return x @ W.T + b


def _leaky(x):
    return jnp.where(x >= 0, x, 0.01 * x)


def _layer_norm(x, w, b, eps=1e-5):
    m = jnp.mean(x, axis=-1, keepdims=True)
    v = jnp.var(x, axis=-1, keepdims=True)
    return (x - m) / jnp.sqrt(v + eps) * w + b


def _graph_norm(x, w, b, ms, eps):
    mean = jnp.mean(x, axis=0)
    out = x - mean * ms
    var = jnp.mean(out ** 2, axis=0)
    return w * out / jnp.sqrt(var + eps) + b


def _sc_degrees(src, dst):
    e_total = src.shape[0]
    per_w = e_total // _NW
    n_chunks = per_w // _K
    per_tile = ((N + _NS - 1) // _NS + 7) // 8 * 8
    num_pad = per_tile * _NS
    mesh = plsc.VectorSubcoreMesh(core_axis_name="c", subcore_axis_name="s")
    zeros = jnp.zeros((_K,), jnp.float32)
    ones = jnp.ones((_K,), jnp.float32)

    chunks = []
    off = 0
    while off < per_tile:
        ln = min(_K, per_tile - off)
        chunks.append((off, ln))
        off += ln

    @functools.partial(
        pl.kernel,
        out_type=jax.ShapeDtypeStruct((_NC * 2 * num_pad,), jnp.float32),
        mesh=mesh,
        scratch_types=[
            pltpu.VMEM((_K,), jnp.int32),
            pltpu.VMEM((_K,), jnp.int32),
            pltpu.VMEM((_K,), jnp.float32),
            pltpu.VMEM((_K,), jnp.float32),
            pltpu.VMEM_SHARED((num_pad,), jnp.float32),
            pltpu.VMEM_SHARED((num_pad,), jnp.float32),
        ],
    )
    def k(src_h, dst_h, zero_h, one_h, out_h, si_v, di_v, zb_v, ones_v,
          accn_s, acce_s):
        cid = lax.axis_index("c")
        sid = lax.axis_index("s")
        wid = cid * _NS + sid
        row0 = sid * per_tile
        pltpu.sync_copy(zero_h, zb_v)
        pltpu.sync_copy(one_h, ones_v)
        for off, ln in chunks:
            pltpu.sync_copy(zb_v.at[pl.ds(0, ln)], accn_s.at[pl.ds(row0 + off, ln)])
            pltpu.sync_copy(zb_v.at[pl.ds(0, ln)], acce_s.at[pl.ds(row0 + off, ln)])
        plsc.subcore_barrier()
        base0 = wid * per_w

        def body(j, carry):
            base = base0 + j * _K
            pltpu.sync_copy(src_h.at[pl.ds(base, _K)], si_v)
            pltpu.sync_copy(dst_h.at[pl.ds(base, _K)], di_v)
            pltpu.sync_copy(ones_v, accn_s.at[si_v], add=True)
            pltpu.sync_copy(ones_v, acce_s.at[di_v], add=True)
            return carry

        lax.fori_loop(0, n_chunks, body, 0)
        plsc.subcore_barrier()
        for off, ln in chunks:
            pltpu.sync_copy(accn_s.at[pl.ds(row0 + off, ln)], zb_v.at[pl.ds(0, ln)])
            pltpu.sync_copy(zb_v.at[pl.ds(0, ln)],
                            out_h.at[pl.ds(cid * 2 * num_pad + row0 + off, ln)])
            pltpu.sync_copy(acce_s.at[pl.ds(row0 + off, ln)], ones_v.at[pl.ds(0, ln)])
            pltpu.sync_copy(ones_v.at[pl.ds(0, ln)],
                            out_h.at[pl.ds(cid * 2 * num_pad + num_pad + row0 + off, ln)])

    out = k(src, dst, zeros, ones)
    return out.reshape(_NC, 2, num_pad)


_MC = 512
_MROWS = 320
_MPAD = 328
_BIG = 3.0e38


def _sc_seg_min2(table_a, table_b, gidx, sidx):
    e_total = gidx.shape[0]
    n_chunks = e_total // _MC
    num_pad = _MROWS * _NW
    groups = _MC // 16
    n_batches = (_MC + 15 + _K - 1) // _K
    mesh = plsc.VectorSubcoreMesh(core_axis_name="c", subcore_axis_name="s")
    inf_rows = jnp.full((_K, H), _BIG, jnp.float32)

    acc_chunks = []
    off = 0
    while off < _MPAD:
        ln = min(_K, _MPAD - off)
        acc_chunks.append((off, ln))
        off += ln
    out_chunks = []
    off = 0
    while off < _MROWS:
        ln = min(_K, _MROWS - off)
        out_chunks.append((off, ln))
        off += ln

    @functools.partial(
        pl.kernel,
        out_type=(jax.ShapeDtypeStruct((num_pad, H), jnp.float32),
                  jax.ShapeDtypeStruct((num_pad, H), jnp.float32)),
        mesh=mesh,
        scratch_types=[
            pltpu.VMEM((_MC,), jnp.int32),
            pltpu.VMEM((_MC,), jnp.int32),
            pltpu.VMEM((_MC + _K,), jnp.int32),
            pltpu.VMEM((_MC + _K,), jnp.int32),
            pltpu.VMEM((_K, H), jnp.float32),
            pltpu.VMEM((_K, H), jnp.float32),
            pltpu.VMEM((_MPAD, H), jnp.float32),
            pltpu.VMEM((_MPAD, H), jnp.float32),
            pltpu.SemaphoreType.DMA,
            pltpu.SemaphoreType.DMA,
        ],
    )
    def k(ta_h, tb_h, gidx_h, sidx_h, inf_h, oa_h, ob_h, dv, sv, md, ms,
          ra_v, rb_v, acc_a, acc_b, sem_a, sem_b):
        cid = lax.axis_index("c")
        sid = lax.axis_index("s")
        wid = cid * _NS + sid
        lo = wid * _MROWS
        for off, ln in acc_chunks:
            pltpu.sync_copy(inf_h.at[pl.ds(0, ln)], acc_a.at[pl.ds(off, ln)])
            pltpu.sync_copy(inf_h.at[pl.ds(0, ln)], acc_b.at[pl.ds(off, ln)])
        z16 = jnp.zeros((16,), jnp.int32)
        for j in range((_MC + _K) // 16):
            ms[pl.ds(j * 16, 16)] = z16
        iota = lax.iota(jnp.int32, 16)

        def chunk_body(ci, carry):
            base = ci * _MC
            pltpu.sync_copy(sidx_h.at[pl.ds(base, _MC)], dv)
            pltpu.sync_copy(gidx_h.at[pl.ds(base, _MC)], sv)

            def scan_group(g, nmatch):
                dvec = dv[pl.ds(g * 16, 16)]
                svec = sv[pl.ds(g * 16, 16)]
                mask = (dvec >= lo) & (dvec < lo + _MROWS)
                p = jnp.where(mask, 1, 0)
                for sh in (1, 2, 4, 8):
                    src_l = jnp.maximum(iota - sh, 0)
                    shifted = p.at[src_l].get(mode="promise_in_bounds")
                    p = p + jnp.where(iota >= sh, shifted, 0)
                cnt = p[15]
                pos = jnp.zeros((16,), jnp.int32)
                tgt = iota + 1
                for step in (8, 4, 2, 1):
                    probe_i = jnp.minimum(pos + (step - 1), 15)
                    v = p.at[probe_i].get(mode="promise_in_bounds")
                    pos = jnp.where(v < tgt, pos + step, pos)
                posc = jnp.minimum(pos, 15)
                cd = (dvec - lo).at[posc].get(mode="promise_in_bounds")
                cs = svec.at[posc].get(mode="promise_in_bounds")
                md[pl.ds(nmatch, 16)] = cd
                ms[pl.ds(nmatch, 16)] = cs
                return nmatch + cnt

            nmatch = lax.fori_loop(0, groups, scan_group, 0)

            for b in range(n_batches):
                @pl.when(nmatch > b * _K)
                def _process():
                    ca = pltpu.async_copy(ta_h.at[ms.at[pl.ds(b * _K, _K)]],
                                          ra_v, sem_a)
                    cb = pltpu.async_copy(tb_h.at[ms.at[pl.ds(b * _K, _K)]],
                                          rb_v, sem_b)
                    ca.wait()
                    cb.wait()

                    def upd(i, c2):
                        d = md[pl.ds(i, 16)][0]
                        r = i - b * _K
                        for cc in range(H // 16):
                            cs_ = pl.ds(cc * 16, 16)
                            acc_a[d, cs_] = jnp.minimum(acc_a[d, cs_],
                                                        ra_v[r, cs_])
                            acc_b[d, cs_] = jnp.minimum(acc_b[d, cs_],
                                                        rb_v[r, cs_])
                        return c2

                    lax.fori_loop(b * _K, jnp.minimum(nmatch, (b + 1) * _K),
                                  upd, 0)
            return carry

        lax.fori_loop(0, n_chunks, chunk_body, 0)
        for off, ln in out_chunks:
            pltpu.sync_copy(acc_a.at[pl.ds(off, ln)], oa_h.at[pl.ds(lo + off, ln)])
            pltpu.sync_copy(acc_b.at[pl.ds(off, ln)], ob_h.at[pl.ds(lo + off, ln)])

    oa, ob = k(table_a, table_b, gidx, sidx, inf_rows)
    return oa[:M], ob[:M]


def _seg_sum(table, gidx, sidx, num_out):
    p = _sc_seg_sum_rows(table, gidx, sidx, num_out)
    return p[0, :num_out] + p[1, :num_out]


def _hgconv(x, src, dst, W, b, deg_n_inv, deg_e_inv):
    xl = x @ W.T
    out_e = _seg_sum(xl, src, dst, M) * deg_e_inv[:, None]
    out_n = _seg_sum(out_e, dst, src, N) * deg_n_inv[:, None]
    return out_n + b


def _hgconv_dual(x, src, dst, W, b, deg_n_inv, deg_e_inv):
    xl = x @ W.T
    out_e = _seg_sum(xl, dst, src, N) * deg_n_inv[:, None]
    out_n = _seg_sum(out_e, src, dst, M) * deg_e_inv[:, None]
    return out_n + b


def _mm_kernel(x_ref, w_ref, b_ref, o_ref):
    o_ref[...] = jnp.dot(x_ref[...], w_ref[...],
                         preferred_element_type=jnp.float32) + b_ref[...]


def _pallas_linear(x, W, b):
    m = x.shape[0]
    return pl.pallas_call(
        _mm_kernel,
        out_shape=jax.ShapeDtypeStruct((m, W.shape[0]), jnp.float32),
    )(x, W.T, b[None, :])


def kernel(x, x_e, edge_index, params):
    p = params
    src = edge_index[0]
    dst = edge_index[1]

    xs = p['x_struct']
    xs = _leaky(_linear(_l2norm(xs), p['in_proj_w'], p['in_proj_b']))
    xn = _leaky(_linear(_l2norm(x), p['n_sem_w'], p['n_sem_b']))
    xe = _leaky(_linear(_l2norm(x_e), p['e_proj_w'], p['e_proj_b']))

    degp = _sc_degrees(src, dst)
    deg_n = degp[0, 0, :N] + degp[1, 0, :N]
    deg_e = degp[0, 1, :M] + degp[1, 1, :M]
    deg_n_inv = jnp.where(deg_n == 0, 0.0, 1.0 / deg_n)
    deg_e_inv = jnp.where(deg_e == 0, 0.0, 1.0 / deg_e)

    xs = _layer_norm(xs, p['n_norm_w'], p['n_norm_b'])
    xs = _leaky(_hgconv(xs, src, dst, p['hg0_w'], p['hg0_b'], deg_n_inv, deg_e_inv)) \
        + _graph_norm(xs, p['gn_s_w'], p['gn_s_b'], p['gn_s_ms'], float(H))
    xn = _graph_norm(xn, p['gn1_w'], p['gn1_b'], p['gn1_ms'], 1e-5)
    xn = _leaky(_hgconv(xn, src, dst, p['hg1_w'], p['hg1_b'], deg_n_inv, deg_e_inv)) \
        + _linear(xn, p['skip1_w'], p['skip1_b'])
    xf = jnp.concatenate([xs, xn], axis=1)
    xf = _layer_norm(xf, p['nf_ln1_w'], p['nf_ln1_b'])
    xf = _leaky(_linear(xf, p['nf_lin_w'], p['nf_lin_b']))
    xf = _layer_norm(xf, p['nf_ln2_w'], p['nf_ln2_b'])
    agg, xa = _sc_seg_min2(xn, xf, src, dst)
    xe = _graph_norm(xe + agg, p['gn2_w'], p['gn2_b'], p['gn2_ms'], 1e-5)
    xe = _leaky(_hgconv_dual(xe, src, dst, p['hg2_w'], p['hg2_b'], deg_n_inv, deg_e_inv)) \
        + _linear(xe, p['skip2_w'], p['skip2_b'])
    xef = jnp.concatenate([xa, xe], axis=1)
    xef = _layer_norm(xef, p['ef_ln1_w'], p['ef_ln1_b'])
    xef = _leaky(_linear(xef, p['ef_lin_w'], p['ef_lin_b']))
    xef = _layer_norm(xef, p['ef_ln2_w'], p['ef_ln2_b'])
    return _pallas_linear(xef, p['out_w'], p['out_b'])


# speedup vs baseline: 1.4105x; 1.4105x over previous
import functools

import jax
import jax.numpy as jnp
from jax import lax
from jax.experimental import pallas as pl
from jax.experimental.pallas import tpu as pltpu
from jax.experimental.pallas import tpu_sc as plsc

N = 10000
M = 10000
E = 320000
H = 128

_NC = 2
_NS = 16
_NW = _NC * _NS
_K = 80


def _sc_seg_sum_rows(table, gidx, sidx, num_out):
    e_total = gidx.shape[0]
    per_w = e_total // _NW
    n_chunks = per_w // _K
    rows_per_tile = ((num_out + _NS - 1) // _NS + 7) // 8 * 8
    num_pad = rows_per_tile * _NS
    mesh = plsc.VectorSubcoreMesh(core_axis_name="c", subcore_axis_name="s")
    zeros = jnp.zeros((_K, H), jnp.float32)

    chunks = []
    off = 0
    while off < rows_per_tile:
        ln = min(_K, rows_per_tile - off)
        chunks.append((off, ln))
        off += ln

    @functools.partial(
        pl.kernel,
        out_type=jax.ShapeDtypeStruct((_NC * num_pad, H), jnp.float32),
        mesh=mesh,
        scratch_types=[
            pltpu.VMEM((_K,), jnp.int32),
            pltpu.VMEM((_K,), jnp.int32),
            pltpu.VMEM((_K, H), jnp.float32),
            pltpu.VMEM_SHARED((num_pad, H), jnp.float32),
            pltpu.SemaphoreType.DMA,
        ],
    )
    def k(table_h, gidx_h, sidx_h, zero_h, out_h, gi_v, si_v, rows_v, acc_s, sem):
        cid = lax.axis_index("c")
        sid = lax.axis_index("s")
        wid = cid * _NS + sid
        row0 = sid * rows_per_tile
        pltpu.sync_copy(zero_h, rows_v)
        for off, ln in chunks:
            pltpu.sync_copy(rows_v.at[pl.ds(0, ln)], acc_s.at[pl.ds(row0 + off, ln)])
        plsc.subcore_barrier()
        base0 = wid * per_w

        def body(j, carry):
            base = base0 + j * _K
            pltpu.sync_copy(gidx_h.at[pl.ds(base, _K)], gi_v)
            pltpu.sync_copy(sidx_h.at[pl.ds(base, _K)], si_v)
            pltpu.async_copy(table_h.at[gi_v], rows_v, sem).wait()
            pltpu.sync_copy(rows_v, acc_s.at[si_v], add=True)
            return carry

        lax.fori_loop(0, n_chunks, body, 0)
        plsc.subcore_barrier()
        out0 = cid * num_pad + row0
        for off, ln in chunks:
            pltpu.sync_copy(acc_s.at[pl.ds(row0 + off, ln)], rows_v.at[pl.ds(0, ln)])
            pltpu.sync_copy(rows_v.at[pl.ds(0, ln)], out_h.at[pl.ds(out0 + off, ln)])

    out = k(table, gidx, sidx, zeros)
    return out.reshape(_NC, num_pad, H)


def _l2norm(x):
    n = jnp.linalg.norm(x, axis=1, keepdims=True)
    return x / jnp.maximum(n, 1e-12)


def _linear(x, W, b):
    return x @ W.T + b


def _leaky(x):
    return jnp.where(x >= 0, x, 0.01 * x)


def _layer_norm(x, w, b, eps=1e-5):
    m = jnp.mean(x, axis=-1, keepdims=True)
    v = jnp.var(x, axis=-1, keepdims=True)
    return (x - m) / jnp.sqrt(v + eps) * w + b


def _graph_norm(x, w, b, ms, eps):
    mean = jnp.mean(x, axis=0)
    out = x - mean * ms
    var = jnp.mean(out ** 2, axis=0)
    return w * out / jnp.sqrt(var + eps) + b


def _sc_degrees(src, dst):
    e_total = src.shape[0]
    per_w = e_total // _NW
    n_chunks = per_w // _K
    per_tile = ((N + _NS - 1) // _NS + 7) // 8 * 8
    num_pad = per_tile * _NS
    mesh = plsc.VectorSubcoreMesh(core_axis_name="c", subcore_axis_name="s")
    zeros = jnp.zeros((_K,), jnp.float32)
    ones = jnp.ones((_K,), jnp.float32)

    chunks = []
    off = 0
    while off < per_tile:
        ln = min(_K, per_tile - off)
        chunks.append((off, ln))
        off += ln

    @functools.partial(
        pl.kernel,
        out_type=jax.ShapeDtypeStruct((_NC * 2 * num_pad,), jnp.float32),
        mesh=mesh,
        scratch_types=[
            pltpu.VMEM((_K,), jnp.int32),
            pltpu.VMEM((_K,), jnp.int32),
            pltpu.VMEM((_K,), jnp.float32),
            pltpu.VMEM((_K,), jnp.float32),
            pltpu.VMEM_SHARED((num_pad,), jnp.float32),
            pltpu.VMEM_SHARED((num_pad,), jnp.float32),
        ],
    )
    def k(src_h, dst_h, zero_h, one_h, out_h, si_v, di_v, zb_v, ones_v,
          accn_s, acce_s):
        cid = lax.axis_index("c")
        sid = lax.axis_index("s")
        wid = cid * _NS + sid
        row0 = sid * per_tile
        pltpu.sync_copy(zero_h, zb_v)
        pltpu.sync_copy(one_h, ones_v)
        for off, ln in chunks:
            pltpu.sync_copy(zb_v.at[pl.ds(0, ln)], accn_s.at[pl.ds(row0 + off, ln)])
            pltpu.sync_copy(zb_v.at[pl.ds(0, ln)], acce_s.at[pl.ds(row0 + off, ln)])
        plsc.subcore_barrier()
        base0 = wid * per_w

        def body(j, carry):
            base = base0 + j * _K
            pltpu.sync_copy(src_h.at[pl.ds(base, _K)], si_v)
            pltpu.sync_copy(dst_h.at[pl.ds(base, _K)], di_v)
            pltpu.sync_copy(ones_v, accn_s.at[si_v], add=True)
            pltpu.sync_copy(ones_v, acce_s.at[di_v], add=True)
            return carry

        lax.fori_loop(0, n_chunks, body, 0)
        plsc.subcore_barrier()
        for off, ln in chunks:
            pltpu.sync_copy(accn_s.at[pl.ds(row0 + off, ln)], zb_v.at[pl.ds(0, ln)])
            pltpu.sync_copy(zb_v.at[pl.ds(0, ln)],
                            out_h.at[pl.ds(cid * 2 * num_pad + row0 + off, ln)])
            pltpu.sync_copy(acce_s.at[pl.ds(row0 + off, ln)], ones_v.at[pl.ds(0, ln)])
            pltpu.sync_copy(ones_v.at[pl.ds(0, ln)],
                            out_h.at[pl.ds(cid * 2 * num_pad + num_pad + row0 + off, ln)])

    out = k(src, dst, zeros, ones)
    return out.reshape(_NC, 2, num_pad)


_MC = 2000
_MROWS = 320
_MPAD = 328
_BIG = 3.0e38


def _sc_seg_min2(table_a, table_b, gidx, sidx):
    e_total = gidx.shape[0]
    n_chunks = e_total // _MC
    num_pad = _MROWS * _NW
    groups = _MC // 16
    n_batches = (_MC + 15 + _K - 1) // _K
    mesh = plsc.VectorSubcoreMesh(core_axis_name="c", subcore_axis_name="s")
    inf_rows = jnp.full((_K, H), _BIG, jnp.float32)

    acc_chunks = []
    off = 0
    while off < _MPAD:
        ln = min(_K, _MPAD - off)
        acc_chunks.append((off, ln))
        off += ln
    out_chunks = []
    off = 0
    while off < _MROWS:
        ln = min(_K, _MROWS - off)
        out_chunks.append((off, ln))
        off += ln

    @functools.partial(
        pl.kernel,
        out_type=(jax.ShapeDtypeStruct((num_pad, H), jnp.float32),
                  jax.ShapeDtypeStruct((num_pad, H), jnp.float32)),
        mesh=mesh,
        scratch_types=[
            pltpu.VMEM((2 * _MC,), jnp.int32),
            pltpu.VMEM((2 * _MC,), jnp.int32),
            pltpu.VMEM((_MC + _K,), jnp.int32),
            pltpu.VMEM((_MC + _K,), jnp.int32),
            pltpu.VMEM((_K, H), jnp.float32),
            pltpu.VMEM((_K, H), jnp.float32),
            pltpu.VMEM((_MPAD, H), jnp.float32),
            pltpu.VMEM((_MPAD, H), jnp.float32),
            pltpu.SemaphoreType.DMA,
            pltpu.SemaphoreType.DMA,
            pltpu.SemaphoreType.DMA,
        ],
    )
    def k(ta_h, tb_h, gidx_h, sidx_h, inf_h, oa_h, ob_h, dvb, svb, md, ms,
          ra_v, rb_v, acc_a, acc_b, sem_i, sem_a, sem_b):
        cid = lax.axis_index("c")
        sid = lax.axis_index("s")
        wid = cid * _NS + sid
        lo = wid * _MROWS
        for off, ln in acc_chunks:
            pltpu.sync_copy(inf_h.at[pl.ds(0, ln)], acc_a.at[pl.ds(off, ln)])
            pltpu.sync_copy(inf_h.at[pl.ds(0, ln)], acc_b.at[pl.ds(off, ln)])
        z16 = jnp.zeros((16,), jnp.int32)
        for j in range((_MC + _K) // 16):
            ms[pl.ds(j * 16, 16)] = z16
        iota = lax.iota(jnp.int32, 16)
        pltpu.async_copy(sidx_h.at[pl.ds(0, _MC)], dvb.at[pl.ds(0, _MC)], sem_i)
        pltpu.async_copy(gidx_h.at[pl.ds(0, _MC)], svb.at[pl.ds(0, _MC)], sem_i)

        def chunk_body(ci, carry):
            par = (ci % 2) * _MC
            pltpu.make_async_copy(sidx_h.at[pl.ds(0, _MC)],
                                  dvb.at[pl.ds(par, _MC)], sem_i).wait()
            pltpu.make_async_copy(gidx_h.at[pl.ds(0, _MC)],
                                  svb.at[pl.ds(par, _MC)], sem_i).wait()

            @pl.when(ci + 1 < n_chunks)
            def _prefetch():
                base2 = (ci + 1) * _MC
                par2 = _MC - par
                pltpu.async_copy(sidx_h.at[pl.ds(base2, _MC)],
                                 dvb.at[pl.ds(par2, _MC)], sem_i)
                pltpu.async_copy(gidx_h.at[pl.ds(base2, _MC)],
                                 svb.at[pl.ds(par2, _MC)], sem_i)

            def scan_group(g, nmatch):
                dvec = dvb[pl.ds(par + g * 16, 16)]
                mask = (dvec >= lo) & (dvec < lo + _MROWS)
                p = jnp.where(mask, 1, 0)
                for sh in (1, 2, 4, 8):
                    src_l = jnp.maximum(iota - sh, 0)
                    shifted = p.at[src_l].get(mode="promise_in_bounds")
                    p = p + jnp.where(iota >= sh, shifted, 0)
                cnt = p[15]

                @pl.when(cnt > 0)
                def _compact():
                    svec = svb[pl.ds(par + g * 16, 16)]
                    pos = jnp.zeros((16,), jnp.int32)
                    tgt = iota + 1
                    for step in (8, 4, 2, 1):
                        probe_i = jnp.minimum(pos + (step - 1), 15)
                        v = p.at[probe_i].get(mode="promise_in_bounds")
                        pos = jnp.where(v < tgt, pos + step, pos)
                    posc = jnp.minimum(pos, 15)
                    cd = (dvec - lo).at[posc].get(mode="promise_in_bounds")
                    cs = svec.at[posc].get(mode="promise_in_bounds")
                    md[pl.ds(nmatch, 16)] = cd
                    ms[pl.ds(nmatch, 16)] = cs

                return nmatch + cnt

            nmatch = lax.fori_loop(0, groups, scan_group, 0)

            for b in range(n_batches):
                @pl.when(nmatch > b * _K)
                def _process():
                    ca = pltpu.async_copy(ta_h.at[ms.at[pl.ds(b * _K, _K)]],
                                          ra_v, sem_a)
                    cb = pltpu.async_copy(tb_h.at[ms.at[pl.ds(b * _K, _K)]],
                                          rb_v, sem_b)
                    ca.wait()
                    cb.wait()

                    def upd(i, c2):
                        d = md[pl.ds(i, 16)][0]
                        r = i - b * _K
                        for cc in range(H // 16):
                            cs_ = pl.ds(cc * 16, 16)
                            acc_a[d, cs_] = jnp.minimum(acc_a[d, cs_],
                                                        ra_v[r, cs_])
                            acc_b[d, cs_] = jnp.minimum(acc_b[d, cs_],
                                                        rb_v[r, cs_])
                        return c2

                    lax.fori_loop(b * _K, jnp.minimum(nmatch, (b + 1) * _K),
                                  upd, 0)
            return carry

        lax.fori_loop(0, n_chunks, chunk_body, 0)
        for off, ln in out_chunks:
            pltpu.sync_copy(acc_a.at[pl.ds(off, ln)], oa_h.at[pl.ds(lo + off, ln)])
            pltpu.sync_copy(acc_b.at[pl.ds(off, ln)], ob_h.at[pl.ds(lo + off, ln)])

    oa, ob = k(table_a, table_b, gidx, sidx, inf_rows)
    return oa[:M], ob[:M]


def _seg_sum(table, gidx, sidx, num_out):
    p = _sc_seg_sum_rows(table, gidx, sidx, num_out)
    return p[0, :num_out] + p[1, :num_out]


def _hgconv(x, src, dst, W, b, deg_n_inv, deg_e_inv):
    xl = x @ W.T
    out_e = _seg_sum(xl, src, dst, M) * deg_e_inv[:, None]
    out_n = _seg_sum(out_e, dst, src, N) * deg_n_inv[:, None]
    return out_n + b


def _hgconv_dual(x, src, dst, W, b, deg_n_inv, deg_e_inv):
    xl = x @ W.T
    out_e = _seg_sum(xl, dst, src, N) * deg_n_inv[:, None]
    out_n = _seg_sum(out_e, src, dst, M) * deg_e_inv[:, None]
    return out_n + b


def _mm_kernel(x_ref, w_ref, b_ref, o_ref):
    o_ref[...] = jnp.dot(x_ref[...], w_ref[...],
                         preferred_element_type=jnp.float32) + b_ref[...]


def _pallas_linear(x, W, b):
    m = x.shape[0]
    return pl.pallas_call(
        _mm_kernel,
        out_shape=jax.ShapeDtypeStruct((m, W.shape[0]), jnp.float32),
    )(x, W.T, b[None, :])


def kernel(x, x_e, edge_index, params):
    p = params
    src = edge_index[0]
    dst = edge_index[1]

    xs = p['x_struct']
    xs = _leaky(_linear(_l2norm(xs), p['in_proj_w'], p['in_proj_b']))
    xn = _leaky(_linear(_l2norm(x), p['n_sem_w'], p['n_sem_b']))
    xe = _leaky(_linear(_l2norm(x_e), p['e_proj_w'], p['e_proj_b']))

    degp = _sc_degrees(src, dst)
    deg_n = degp[0, 0, :N] + degp[1, 0, :N]
    deg_e = degp[0, 1, :M] + degp[1, 1, :M]
    deg_n_inv = jnp.where(deg_n == 0, 0.0, 1.0 / deg_n)
    deg_e_inv = jnp.where(deg_e == 0, 0.0, 1.0 / deg_e)

    xs = _layer_norm(xs, p['n_norm_w'], p['n_norm_b'])
    xs = _leaky(_hgconv(xs, src, dst, p['hg0_w'], p['hg0_b'], deg_n_inv, deg_e_inv)) \
        + _graph_norm(xs, p['gn_s_w'], p['gn_s_b'], p['gn_s_ms'], float(H))
    xn = _graph_norm(xn, p['gn1_w'], p['gn1_b'], p['gn1_ms'], 1e-5)
    xn = _leaky(_hgconv(xn, src, dst, p['hg1_w'], p['hg1_b'], deg_n_inv, deg_e_inv)) \
        + _linear(xn, p['skip1_w'], p['skip1_b'])
    xf = jnp.concatenate([xs, xn], axis=1)
    xf = _layer_norm(xf, p['nf_ln1_w'], p['nf_ln1_b'])
    xf = _leaky(_linear(xf, p['nf_lin_w'], p['nf_lin_b']))
    xf = _layer_norm(xf, p['nf_ln2_w'], p['nf_ln2_b'])
    agg, xa = _sc_seg_min2(xn, xf, src, dst)
    xe = _graph_norm(xe + agg, p['gn2_w'], p['gn2_b'], p['gn2_ms'], 1e-5)
    xe = _leaky(_hgconv_dual(xe, src, dst, p['hg2_w'], p['hg2_b'], deg_n_inv, deg_e_inv)) \
        + _linear(xe, p['skip2_w'], p['skip2_b'])
    xef = jnp.concatenate([xa, xe], axis=1)
    xef = _layer_norm(xef, p['ef_ln1_w'], p['ef_ln1_b'])
    xef = _leaky(_linear(xef, p['ef_lin_w'], p['ef_lin_b']))
    xef = _layer_norm(xef, p['ef_ln2_w'], p['ef_ln2_b'])
    return _pallas_linear(xef, p['out_w'], p['out_b'])


# speedup vs baseline: 1.8256x; 1.2943x over previous
import functools

import jax
import jax.numpy as jnp
from jax import lax
from jax.experimental import pallas as pl
from jax.experimental.pallas import tpu as pltpu
from jax.experimental.pallas import tpu_sc as plsc

N = 10000
M = 10000
E = 320000
H = 128

_NC = 2
_NS = 16
_NW = _NC * _NS
_K = 80


def _sc_seg_sum_rows(table, gidx, sidx, num_out):
    e_total = gidx.shape[0]
    per_w = e_total // _NW
    n_chunks = per_w // _K
    rows_per_tile = ((num_out + _NS - 1) // _NS + 7) // 8 * 8
    num_pad = rows_per_tile * _NS
    mesh = plsc.VectorSubcoreMesh(core_axis_name="c", subcore_axis_name="s")
    zeros = jnp.zeros((_K, H), jnp.float32)

    chunks = []
    off = 0
    while off < rows_per_tile:
        ln = min(_K, rows_per_tile - off)
        chunks.append((off, ln))
        off += ln

    @functools.partial(
        pl.kernel,
        out_type=jax.ShapeDtypeStruct((_NC * num_pad, H), jnp.float32),
        mesh=mesh,
        scratch_types=[
            pltpu.VMEM((2 * _K,), jnp.int32),
            pltpu.VMEM((2 * _K,), jnp.int32),
            pltpu.VMEM((2 * _K, H), jnp.float32),
            pltpu.VMEM_SHARED((num_pad, H), jnp.float32),
            pltpu.SemaphoreType.DMA,
            pltpu.SemaphoreType.DMA,
        ],
    )
    def k(table_h, gidx_h, sidx_h, zero_h, out_h, gi_v, si_v, rows_v, acc_s,
          sem_i, sem_g):
        cid = lax.axis_index("c")
        sid = lax.axis_index("s")
        wid = cid * _NS + sid
        row0 = sid * rows_per_tile
        pltpu.sync_copy(zero_h, rows_v.at[pl.ds(0, _K)])
        for off, ln in chunks:
            pltpu.sync_copy(rows_v.at[pl.ds(0, ln)], acc_s.at[pl.ds(row0 + off, ln)])
        plsc.subcore_barrier()
        base0 = wid * per_w
        pltpu.sync_copy(gidx_h.at[pl.ds(base0, _K)], gi_v.at[pl.ds(0, _K)])
        pltpu.sync_copy(sidx_h.at[pl.ds(base0, _K)], si_v.at[pl.ds(0, _K)])
        pltpu.async_copy(table_h.at[gi_v.at[pl.ds(0, _K)]],
                         rows_v.at[pl.ds(0, _K)], sem_g)
        if n_chunks > 1:
            pltpu.async_copy(gidx_h.at[pl.ds(base0 + _K, _K)],
                             gi_v.at[pl.ds(_K, _K)], sem_i)
            pltpu.async_copy(sidx_h.at[pl.ds(base0 + _K, _K)],
                             si_v.at[pl.ds(_K, _K)], sem_i)

        def body(j, carry):
            par = (j % 2) * _K
            par2 = _K - par
            pltpu.make_async_copy(table_h.at[gi_v.at[pl.ds(0, _K)]],
                                  rows_v.at[pl.ds(par, _K)], sem_g).wait()

            @pl.when(j + 1 < n_chunks)
            def _next():
                pltpu.make_async_copy(gidx_h.at[pl.ds(0, _K)],
                                      gi_v.at[pl.ds(par2, _K)], sem_i).wait()
                pltpu.make_async_copy(sidx_h.at[pl.ds(0, _K)],
                                      si_v.at[pl.ds(par2, _K)], sem_i).wait()
                pltpu.async_copy(table_h.at[gi_v.at[pl.ds(par2, _K)]],
                                 rows_v.at[pl.ds(par2, _K)], sem_g)

            pltpu.sync_copy(rows_v.at[pl.ds(par, _K)], acc_s.at[si_v.at[pl.ds(par, _K)]],
                            add=True)

            @pl.when(j + 2 < n_chunks)
            def _prefetch():
                base2 = base0 + (j + 2) * _K
                pltpu.async_copy(gidx_h.at[pl.ds(base2, _K)],
                                 gi_v.at[pl.ds(par, _K)], sem_i)
                pltpu.async_copy(sidx_h.at[pl.ds(base2, _K)],
                                 si_v.at[pl.ds(par, _K)], sem_i)

            return carry

        lax.fori_loop(0, n_chunks, body, 0)
        plsc.subcore_barrier()
        out0 = cid * num_pad + row0
        for off, ln in chunks:
            pltpu.sync_copy(acc_s.at[pl.ds(row0 + off, ln)], rows_v.at[pl.ds(0, ln)])
            pltpu.sync_copy(rows_v.at[pl.ds(0, ln)], out_h.at[pl.ds(out0 + off, ln)])

    out = k(table, gidx, sidx, zeros)
    return out.reshape(_NC, num_pad, H)


def _l2norm(x):
    n = jnp.linalg.norm(x, axis=1, keepdims=True)
    return x / jnp.maximum(n, 1e-12)


def _linear(x, W, b):
    return x @ W.T + b


def _leaky(x):
    return jnp.where(x >= 0, x, 0.01 * x)


def _layer_norm(x, w, b, eps=1e-5):
    m = jnp.mean(x, axis=-1, keepdims=True)
    v = jnp.var(x, axis=-1, keepdims=True)
    return (x - m) / jnp.sqrt(v + eps) * w + b


def _graph_norm(x, w, b, ms, eps):
    mean = jnp.mean(x, axis=0)
    out = x - mean * ms
    var = jnp.mean(out ** 2, axis=0)
    return w * out / jnp.sqrt(var + eps) + b


def _sc_degrees(src, dst):
    e_total = src.shape[0]
    per_w = e_total // _NW
    n_chunks = per_w // _K
    per_tile = ((N + _NS - 1) // _NS + 7) // 8 * 8
    num_pad = per_tile * _NS
    mesh = plsc.VectorSubcoreMesh(core_axis_name="c", subcore_axis_name="s")
    zeros = jnp.zeros((_K,), jnp.float32)
    ones = jnp.ones((_K,), jnp.float32)

    chunks = []
    off = 0
    while off < per_tile:
        ln = min(_K, per_tile - off)
        chunks.append((off, ln))
        off += ln

    @functools.partial(
        pl.kernel,
        out_type=jax.ShapeDtypeStruct((_NC * 2 * num_pad,), jnp.float32),
        mesh=mesh,
        scratch_types=[
            pltpu.VMEM((_K,), jnp.int32),
            pltpu.VMEM((_K,), jnp.int32),
            pltpu.VMEM((_K,), jnp.float32),
            pltpu.VMEM((_K,), jnp.float32),
            pltpu.VMEM_SHARED((num_pad,), jnp.float32),
            pltpu.VMEM_SHARED((num_pad,), jnp.float32),
        ],
    )
    def k(src_h, dst_h, zero_h, one_h, out_h, si_v, di_v, zb_v, ones_v,
          accn_s, acce_s):
        cid = lax.axis_index("c")
        sid = lax.axis_index("s")
        wid = cid * _NS + sid
        row0 = sid * per_tile
        pltpu.sync_copy(zero_h, zb_v)
        pltpu.sync_copy(one_h, ones_v)
        for off, ln in chunks:
            pltpu.sync_copy(zb_v.at[pl.ds(0, ln)], accn_s.at[pl.ds(row0 + off, ln)])
            pltpu.sync_copy(zb_v.at[pl.ds(0, ln)], acce_s.at[pl.ds(row0 + off, ln)])
        plsc.subcore_barrier()
        base0 = wid * per_w

        def body(j, carry):
            base = base0 + j * _K
            pltpu.sync_copy(src_h.at[pl.ds(base, _K)], si_v)
            pltpu.sync_copy(dst_h.at[pl.ds(base, _K)], di_v)
            pltpu.sync_copy(ones_v, accn_s.at[si_v], add=True)
            pltpu.sync_copy(ones_v, acce_s.at[di_v], add=True)
            return carry

        lax.fori_loop(0, n_chunks, body, 0)
        plsc.subcore_barrier()
        for off, ln in chunks:
            pltpu.sync_copy(accn_s.at[pl.ds(row0 + off, ln)], zb_v.at[pl.ds(0, ln)])
            pltpu.sync_copy(zb_v.at[pl.ds(0, ln)],
                            out_h.at[pl.ds(cid * 2 * num_pad + row0 + off, ln)])
            pltpu.sync_copy(acce_s.at[pl.ds(row0 + off, ln)], ones_v.at[pl.ds(0, ln)])
            pltpu.sync_copy(ones_v.at[pl.ds(0, ln)],
                            out_h.at[pl.ds(cid * 2 * num_pad + num_pad + row0 + off, ln)])

    out = k(src, dst, zeros, ones)
    return out.reshape(_NC, 2, num_pad)


_MC = 2000
_MROWS = 320
_MPAD = 328
_BIG = 3.0e38


def _sc_seg_min2(table_a, table_b, gidx, sidx):
    e_total = gidx.shape[0]
    n_chunks = e_total // _MC
    num_pad = _MROWS * _NW
    groups = _MC // 16
    n_batches = (_MC + 15 + _K - 1) // _K
    mesh = plsc.VectorSubcoreMesh(core_axis_name="c", subcore_axis_name="s")
    inf_rows = jnp.full((_K, H), _BIG, jnp.float32)

    acc_chunks = []
    off = 0
    while off < _MPAD:
        ln = min(_K, _MPAD - off)
        acc_chunks.append((off, ln))
        off += ln
    out_chunks = []
    off = 0
    while off < _MROWS:
        ln = min(_K, _MROWS - off)
        out_chunks.append((off, ln))
        off += ln

    @functools.partial(
        pl.kernel,
        out_type=(jax.ShapeDtypeStruct((num_pad, H), jnp.float32),
                  jax.ShapeDtypeStruct((num_pad, H), jnp.float32)),
        mesh=mesh,
        scratch_types=[
            pltpu.VMEM((2 * _MC,), jnp.int32),
            pltpu.VMEM((2 * _MC,), jnp.int32),
            pltpu.VMEM((_MC + _K,), jnp.int32),
            pltpu.VMEM((_MC + _K,), jnp.int32),
            pltpu.VMEM((_K, H), jnp.float32),
            pltpu.VMEM((_K, H), jnp.float32),
            pltpu.VMEM((_MPAD, H), jnp.float32),
            pltpu.VMEM((_MPAD, H), jnp.float32),
            pltpu.SemaphoreType.DMA,
            pltpu.SemaphoreType.DMA,
            pltpu.SemaphoreType.DMA,
        ],
    )
    def k(ta_h, tb_h, gidx_h, sidx_h, inf_h, oa_h, ob_h, dvb, svb, md, ms,
          ra_v, rb_v, acc_a, acc_b, sem_i, sem_a, sem_b):
        cid = lax.axis_index("c")
        sid = lax.axis_index("s")
        wid = cid * _NS + sid
        lo = wid * _MROWS
        for off, ln in acc_chunks:
            pltpu.sync_copy(inf_h.at[pl.ds(0, ln)], acc_a.at[pl.ds(off, ln)])
            pltpu.sync_copy(inf_h.at[pl.ds(0, ln)], acc_b.at[pl.ds(off, ln)])
        z16 = jnp.zeros((16,), jnp.int32)
        for j in range((_MC + _K) // 16):
            ms[pl.ds(j * 16, 16)] = z16
        iota = lax.iota(jnp.int32, 16)
        pltpu.async_copy(sidx_h.at[pl.ds(0, _MC)], dvb.at[pl.ds(0, _MC)], sem_i)
        pltpu.async_copy(gidx_h.at[pl.ds(0, _MC)], svb.at[pl.ds(0, _MC)], sem_i)

        def chunk_body(ci, carry):
            par = (ci % 2) * _MC
            pltpu.make_async_copy(sidx_h.at[pl.ds(0, _MC)],
                                  dvb.at[pl.ds(par, _MC)], sem_i).wait()
            pltpu.make_async_copy(gidx_h.at[pl.ds(0, _MC)],
                                  svb.at[pl.ds(par, _MC)], sem_i).wait()

            @pl.when(ci + 1 < n_chunks)
            def _prefetch():
                base2 = (ci + 1) * _MC
                par2 = _MC - par
                pltpu.async_copy(sidx_h.at[pl.ds(base2, _MC)],
                                 dvb.at[pl.ds(par2, _MC)], sem_i)
                pltpu.async_copy(gidx_h.at[pl.ds(base2, _MC)],
                                 svb.at[pl.ds(par2, _MC)], sem_i)

            def scan_group(g, nmatch):
                dvec = dvb[pl.ds(par + g * 16, 16)]
                mask = (dvec >= lo) & (dvec < lo + _MROWS)
                p = jnp.where(mask, 1, 0)
                for sh in (1, 2, 4, 8):
                    src_l = jnp.maximum(iota - sh, 0)
                    shifted = p.at[src_l].get(mode="promise_in_bounds")
                    p = p + jnp.where(iota >= sh, shifted, 0)
                cnt = p[15]

                @pl.when(cnt > 0)
                def _compact():
                    svec = svb[pl.ds(par + g * 16, 16)]
                    pos = jnp.zeros((16,), jnp.int32)
                    tgt = iota + 1
                    for step in (8, 4, 2, 1):
                        probe_i = jnp.minimum(pos + (step - 1), 15)
                        v = p.at[probe_i].get(mode="promise_in_bounds")
                        pos = jnp.where(v < tgt, pos + step, pos)
                    posc = jnp.minimum(pos, 15)
                    cd = (dvec - lo).at[posc].get(mode="promise_in_bounds")
                    cs = svec.at[posc].get(mode="promise_in_bounds")
                    md[pl.ds(nmatch, 16)] = cd
                    ms[pl.ds(nmatch, 16)] = cs

                return nmatch + cnt

            nmatch = lax.fori_loop(0, groups, scan_group, 0)

            for b in range(n_batches):
                @pl.when(nmatch > b * _K)
                def _process():
                    ca = pltpu.async_copy(ta_h.at[ms.at[pl.ds(b * _K, _K)]],
                                          ra_v, sem_a)
                    cb = pltpu.async_copy(tb_h.at[ms.at[pl.ds(b * _K, _K)]],
                                          rb_v, sem_b)
                    ca.wait()
                    cb.wait()

                    def upd(i, c2):
                        d = md[pl.ds(i, 16)][0]
                        r = i - b * _K
                        for cc in range(H // 16):
                            cs_ = pl.ds(cc * 16, 16)
                            acc_a[d, cs_] = jnp.minimum(acc_a[d, cs_],
                                                        ra_v[r, cs_])
                            acc_b[d, cs_] = jnp.minimum(acc_b[d, cs_],
                                                        rb_v[r, cs_])
                        return c2

                    lax.fori_loop(b * _K, jnp.minimum(nmatch, (b + 1) * _K),
                                  upd, 0)
            return carry

        lax.fori_loop(0, n_chunks, chunk_body, 0)
        for off, ln in out_chunks:
            pltpu.sync_copy(acc_a.at[pl.ds(off, ln)], oa_h.at[pl.ds(lo + off, ln)])
            pltpu.sync_copy(acc_b.at[pl.ds(off, ln)], ob_h.at[pl.ds(lo + off, ln)])

    oa, ob = k(table_a, table_b, gidx, sidx, inf_rows)
    return oa[:M], ob[:M]


def _seg_sum(table, gidx, sidx, num_out):
    p = _sc_seg_sum_rows(table, gidx, sidx, num_out)
    return p[0, :num_out] + p[1, :num_out]


def _hgconv(x, src, dst, W, b, deg_n_inv, deg_e_inv):
    xl = x @ W.T
    out_e = _seg_sum(xl, src, dst, M) * deg_e_inv[:, None]
    out_n = _seg_sum(out_e, dst, src, N) * deg_n_inv[:, None]
    return out_n + b


def _hgconv_dual(x, src, dst, W, b, deg_n_inv, deg_e_inv):
    xl = x @ W.T
    out_e = _seg_sum(xl, dst, src, N) * deg_n_inv[:, None]
    out_n = _seg_sum(out_e, src, dst, M) * deg_e_inv[:, None]
    return out_n + b


def _mm_kernel(x_ref, w_ref, b_ref, o_ref):
    o_ref[...] = jnp.dot(x_ref[...], w_ref[...],
                         preferred_element_type=jnp.float32) + b_ref[...]


def _pallas_linear(x, W, b):
    m = x.shape[0]
    return pl.pallas_call(
        _mm_kernel,
        out_shape=jax.ShapeDtypeStruct((m, W.shape[0]), jnp.float32),
    )(x, W.T, b[None, :])


def kernel(x, x_e, edge_index, params):
    p = params
    src = edge_index[0]
    dst = edge_index[1]

    xs = p['x_struct']
    xs = _leaky(_linear(_l2norm(xs), p['in_proj_w'], p['in_proj_b']))
    xn = _leaky(_linear(_l2norm(x), p['n_sem_w'], p['n_sem_b']))
    xe = _leaky(_linear(_l2norm(x_e), p['e_proj_w'], p['e_proj_b']))

    degp = _sc_degrees(src, dst)
    deg_n = degp[0, 0, :N] + degp[1, 0, :N]
    deg_e = degp[0, 1, :M] + degp[1, 1, :M]
    deg_n_inv = jnp.where(deg_n == 0, 0.0, 1.0 / deg_n)
    deg_e_inv = jnp.where(deg_e == 0, 0.0, 1.0 / deg_e)

    xs = _layer_norm(xs, p['n_norm_w'], p['n_norm_b'])
    xs = _leaky(_hgconv(xs, src, dst, p['hg0_w'], p['hg0_b'], deg_n_inv, deg_e_inv)) \
        + _graph_norm(xs, p['gn_s_w'], p['gn_s_b'], p['gn_s_ms'], float(H))
    xn = _graph_norm(xn, p['gn1_w'], p['gn1_b'], p['gn1_ms'], 1e-5)
    xn = _leaky(_hgconv(xn, src, dst, p['hg1_w'], p['hg1_b'], deg_n_inv, deg_e_inv)) \
        + _linear(xn, p['skip1_w'], p['skip1_b'])
    xf = jnp.concatenate([xs, xn], axis=1)
    xf = _layer_norm(xf, p['nf_ln1_w'], p['nf_ln1_b'])
    xf = _leaky(_linear(xf, p['nf_lin_w'], p['nf_lin_b']))
    xf = _layer_norm(xf, p['nf_ln2_w'], p['nf_ln2_b'])
    agg, xa = _sc_seg_min2(xn, xf, src, dst)
    xe = _graph_norm(xe + agg, p['gn2_w'], p['gn2_b'], p['gn2_ms'], 1e-5)
    xe = _leaky(_hgconv_dual(xe, src, dst, p['hg2_w'], p['hg2_b'], deg_n_inv, deg_e_inv)) \
        + _linear(xe, p['skip2_w'], p['skip2_b'])
    xef = jnp.concatenate([xa, xe], axis=1)
    xef = _layer_norm(xef, p['ef_ln1_w'], p['ef_ln1_b'])
    xef = _leaky(_linear(xef, p['ef_lin_w'], p['ef_lin_b']))
    xef = _layer_norm(xef, p['ef_ln2_w'], p['ef_ln2_b'])
    return _pallas_linear(xef, p['out_w'], p['out_b'])
